# baseline (speedup 1.0000x reference)
import functools

import jax
import jax.numpy as jnp
from jax import lax
from jax.experimental import pallas as pl
from jax.experimental.pallas import tpu as pltpu
from jax.experimental.pallas import tpu_sc as plsc

_TOPK = 5
_ROWS = 128
_COLS = 32768
_LANES = 16
_GATHER_WORKERS = 8
_EXT = 128
_XGRP = 64
_BLK = 8192
_NBLK = _COLS // _BLK


def _extract_body(lab_ref, *refs):
    del lab_ref
    out_ref = refs[-1]
    for i in range(_XGRP):
        out_ref[pl.ds(i * _EXT, _EXT)] = refs[i][pl.ds(i % 8, 1), :].reshape(_EXT)


def _extract_in_spec(i):
    return pl.BlockSpec(
        (8, _EXT),
        lambda g, lab: (g * (_XGRP // 8) + i // 8, lab[g * _XGRP + i] // _EXT))


_extract_call = pl.pallas_call(
    _extract_body,
    grid_spec=pltpu.PrefetchScalarGridSpec(
        num_scalar_prefetch=1,
        grid=(_ROWS // _XGRP,),
        in_specs=[_extract_in_spec(i) for i in range(_XGRP)],
        out_specs=pl.BlockSpec((_XGRP * _EXT,), lambda g, lab: (g,)),
    ),
    out_shape=jax.ShapeDtypeStruct((_ROWS * _EXT,), jnp.float32),
)


def _sc_gather_body(table, labels, out, idx_v, val_v, sem):
    wid = lax.axis_index("s") * 2 + lax.axis_index("c")

    @pl.when(wid < _GATHER_WORKERS)
    def _():
        base = wid * _LANES
        pltpu.sync_copy(labels.at[pl.ds(base, _LANES)], idx_v)
        rows = lax.iota(jnp.int32, _LANES) + base
        flat = rows * _EXT + (idx_v[...] & (_EXT - 1))
        pltpu.async_copy(table.at[flat], val_v, sem).wait()
        pltpu.sync_copy(val_v, out.at[pl.ds(base, _LANES)])


def _make_sc_gather():
    return functools.partial(
        pl.kernel,
        mesh=plsc.VectorSubcoreMesh(core_axis_name="c", subcore_axis_name="s"),
        out_type=jax.ShapeDtypeStruct((_ROWS,), jnp.float32),
        scratch_types=[
            pltpu.VMEM((_LANES,), jnp.int32),
            pltpu.VMEM((_LANES,), jnp.float32),
            pltpu.SemaphoreType.DMA,
        ],
    )(_sc_gather_body)


def _count_body(v_ref, lab_ref, s_ref, out_ref, acc_ref, vb_ref, lb_ref):
    j = pl.program_id(0)

    @pl.when(j == 0)
    def _():
        acc_ref[...] = jnp.zeros_like(acc_ref)
        vb_ref[...] = v_ref[...].reshape(_ROWS, 1)
        lb_ref[...] = lab_ref[...].reshape(_ROWS, 1)

    s = s_ref[...]
    v = vb_ref[...]
    lab = lb_ref[...]
    col = lax.broadcasted_iota(jnp.int32, s.shape, 1) + j * _BLK
    ahead = (s > v) | ((s == v) & (col < lab))
    acc_ref[...] += jnp.sum(ahead.astype(jnp.int32), axis=1, keepdims=True)

    @pl.when(j == _NBLK - 1)
    def _():
        hits = (acc_ref[...] < _TOPK).astype(jnp.float32)
        out_ref[...] = (jnp.sum(hits) / _ROWS).reshape(1, 1)


_count_call = pl.pallas_call(
    _count_body,
    grid=(_NBLK,),
    in_specs=[
        pl.BlockSpec((_ROWS,), lambda j: (0,)),
        pl.BlockSpec((_ROWS,), lambda j: (0,)),
        pl.BlockSpec((_ROWS, _BLK), lambda j: (0, j)),
    ],
    out_specs=pl.BlockSpec((1, 1), lambda j: (0, 0)),
    out_shape=jax.ShapeDtypeStruct((1, 1), jnp.float32),
    scratch_shapes=[
        pltpu.VMEM((_ROWS, 1), jnp.int32),
        pltpu.VMEM((_ROWS, 1), jnp.float32),
        pltpu.VMEM((_ROWS, 1), jnp.int32),
    ],
)


def kernel(scores, labels):
    labels = labels.astype(jnp.int32)
    staged = _extract_call(labels, *([scores] * _XGRP))
    v = _make_sc_gather()(staged, labels)
    out = _count_call(v, labels, scores)
    return out[0, 0]


# speedup vs baseline: 1.1495x; 1.1495x over previous
import functools

import jax
import jax.numpy as jnp
from jax import lax
from jax.experimental import pallas as pl
from jax.experimental.pallas import tpu as pltpu
from jax.experimental.pallas import tpu_sc as plsc

_TOPK = 5
_ROWS = 128
_COLS = 32768
_LANES = 16
_GATHER_WORKERS = 8
_EXT = 128
_XGRP = 128
_BLK = 8192
_NBLK = _COLS // _BLK


def _extract_body(lab_ref, *refs):
    del lab_ref
    out_ref = refs[-1]
    for i in range(_XGRP):
        out_ref[pl.ds(i * _EXT, _EXT)] = refs[i][pl.ds(i % 8, 1), :].reshape(_EXT)


def _extract_in_spec(i):
    return pl.BlockSpec(
        (8, _EXT),
        lambda g, lab: (g * (_XGRP // 8) + i // 8, lab[g * _XGRP + i] // _EXT))


_extract_call = pl.pallas_call(
    _extract_body,
    grid_spec=pltpu.PrefetchScalarGridSpec(
        num_scalar_prefetch=1,
        grid=(_ROWS // _XGRP,),
        in_specs=[_extract_in_spec(i) for i in range(_XGRP)],
        out_specs=pl.BlockSpec((_XGRP * _EXT,), lambda g, lab: (g,)),
    ),
    out_shape=jax.ShapeDtypeStruct((_ROWS * _EXT,), jnp.float32),
)


def _sc_gather_body(table, labels, out, idx_v, val_v, sem):
    wid = lax.axis_index("s") * 2 + lax.axis_index("c")

    @pl.when(wid < _GATHER_WORKERS)
    def _():
        base = wid * _LANES
        pltpu.sync_copy(labels.at[pl.ds(base, _LANES)], idx_v)
        rows = lax.iota(jnp.int32, _LANES) + base
        flat = rows * _EXT + (idx_v[...] & (_EXT - 1))
        pltpu.async_copy(table.at[flat], val_v, sem).wait()
        pltpu.sync_copy(val_v, out.at[pl.ds(base, _LANES)])


def _make_sc_gather():
    return functools.partial(
        pl.kernel,
        mesh=plsc.VectorSubcoreMesh(core_axis_name="c", subcore_axis_name="s"),
        out_type=jax.ShapeDtypeStruct((_ROWS,), jnp.float32),
        scratch_types=[
            pltpu.VMEM((_LANES,), jnp.int32),
            pltpu.VMEM((_LANES,), jnp.float32),
            pltpu.SemaphoreType.DMA,
        ],
    )(_sc_gather_body)


def _count_body(v_ref, lab_ref, s_ref, out_ref, acc_ref, vb_ref, lb_ref):
    j = pl.program_id(0)

    @pl.when(j == 0)
    def _():
        acc_ref[...] = jnp.zeros_like(acc_ref)
        vb_ref[...] = v_ref[...].reshape(_ROWS, 1)
        lb_ref[...] = lab_ref[...].reshape(_ROWS, 1)

    s = s_ref[...]
    v = vb_ref[...]
    lab = lb_ref[...]
    col = lax.broadcasted_iota(jnp.int32, s.shape, 1) + j * _BLK
    ahead = (s > v) | ((s == v) & (col < lab))
    acc_ref[...] += jnp.sum(ahead.astype(jnp.int32), axis=1, keepdims=True)

    @pl.when(j == _NBLK - 1)
    def _():
        hits = (acc_ref[...] < _TOPK).astype(jnp.float32)
        out_ref[...] = (jnp.sum(hits) / _ROWS).reshape(1, 1)


_count_call = pl.pallas_call(
    _count_body,
    grid=(_NBLK,),
    in_specs=[
        pl.BlockSpec((_ROWS,), lambda j: (0,)),
        pl.BlockSpec((_ROWS,), lambda j: (0,)),
        pl.BlockSpec((_ROWS, _BLK), lambda j: (0, j)),
    ],
    out_specs=pl.BlockSpec((1, 1), lambda j: (0, 0)),
    out_shape=jax.ShapeDtypeStruct((1, 1), jnp.float32),
    scratch_shapes=[
        pltpu.VMEM((_ROWS, 1), jnp.int32),
        pltpu.VMEM((_ROWS, 1), jnp.float32),
        pltpu.VMEM((_ROWS, 1), jnp.int32),
    ],
)


def kernel(scores, labels):
    labels = labels.astype(jnp.int32)
    staged = _extract_call(labels, *([scores] * _XGRP))
    v = _make_sc_gather()(staged, labels)
    out = _count_call(v, labels, scores)
    return out[0, 0]
